# baseline (speedup 1.0000x reference)
import functools

import jax
import jax.numpy as jnp
from jax import lax
from jax.experimental import pallas as pl
from jax.experimental.pallas import tpu as pltpu
from jax.experimental.pallas import tpu_sc as plsc

NE = 8
DM = 768
DF = 3072
T = 2048
BM = 256
G = 23
R = 6144
NC = 2
NS = 16
NW = NC * NS
AUX_COEF = 0.01
Z_COEF = 0.001


def _router_body(x_ref, rw_ref, rows_ref, wp_ref, counts_ref, aux_ref, z_ref):
    x = x_ref[...]
    rw = rw_ref[...]
    l = lax.dot_general(x, rw, (((1,), (1,)), ((), ())),
                        preferred_element_type=jnp.float32)
    iota_e = lax.broadcasted_iota(jnp.int32, (T, NE), 1)
    m1 = jnp.max(l, axis=1, keepdims=True)
    i1 = jnp.min(jnp.where(l == m1, iota_e, NE), axis=1, keepdims=True)
    one1 = (iota_e == i1)
    lm = jnp.where(one1, -jnp.inf, l)
    m2 = jnp.max(lm, axis=1, keepdims=True)
    i2 = jnp.min(jnp.where(lm == m2, iota_e, NE), axis=1, keepdims=True)
    one2 = (iota_e == i2)
    w0 = 1.0 / (1.0 + jnp.exp(m2 - m1))
    w1v = 1.0 - w0
    A = one1.astype(jnp.float32) + one2.astype(jnp.float32)
    r_i = lax.broadcasted_iota(jnp.int32, (T, T), 0)
    c_i = lax.broadcasted_iota(jnp.int32, (T, T), 1)
    tri = (c_i < r_i).astype(jnp.float32)
    S = lax.dot_general(tri, A, (((1,), (0,)), ((), ())),
                        preferred_element_type=jnp.float32)
    counts = jnp.sum(A, axis=0, keepdims=True)
    pc = jnp.ceil(counts * (1.0 / BM)) * BM
    e_r = lax.broadcasted_iota(jnp.int32, (NE, NE), 0)
    e_c = lax.broadcasted_iota(jnp.int32, (NE, NE), 1)
    tri8 = (e_r < e_c).astype(jnp.float32)
    off = lax.dot_general(pc, tri8, (((1,), (0,)), ((), ())),
                          preferred_element_type=jnp.float32)
    off0 = jnp.sum(off * one1, axis=1, keepdims=True)
    off1 = jnp.sum(off * one2, axis=1, keepdims=True)
    pos0 = jnp.sum(S * one1, axis=1, keepdims=True)
    pos1 = jnp.sum(S * one2, axis=1, keepdims=True)
    row0 = (off0 + pos0).astype(jnp.int32)
    row1 = (off1 + pos1).astype(jnp.int32)
    rows_ref[...] = jnp.concatenate([row0, row1], axis=1)
    wp_ref[...] = jnp.concatenate([w0, w1v], axis=1)
    counts_ref[...] = counts
    p = jnp.exp(l - m1)
    p = p / jnp.sum(p, axis=1, keepdims=True)
    imp = jnp.mean(p, axis=0, keepdims=True)
    load = jnp.mean(one1.astype(jnp.float32), axis=0, keepdims=True)
    aux_ref[...] = (NE * AUX_COEF) * jnp.sum(imp * load, keepdims=True).reshape(1, 1)
    z_ref[...] = Z_COEF * jnp.mean(l * l, keepdims=True).reshape(1, 1)


def _router(x2, rw):
    return pl.pallas_call(
        _router_body,
        out_shape=[
            jax.ShapeDtypeStruct((T, 2), jnp.int32),
            jax.ShapeDtypeStruct((T, 2), jnp.float32),
            jax.ShapeDtypeStruct((1, NE), jnp.float32),
            jax.ShapeDtypeStruct((1, 1), jnp.float32),
            jax.ShapeDtypeStruct((1, 1), jnp.float32),
        ],
        interpret=False,
    )(x2, rw)


def _ffn_body(be_ref, act_ref, xg_ref, w1_ref, w3_ref, w2_ref, wr_ref, y_ref):
    j = pl.program_id(0)

    @pl.when(act_ref[j] > 0)
    def _():
        xb = xg_ref[...]
        g = lax.dot_general(xb, w1_ref[0], (((1,), (0,)), ((), ())),
                            preferred_element_type=jnp.float32)
        u = lax.dot_general(xb, w3_ref[0], (((1,), (0,)), ((), ())),
                            preferred_element_type=jnp.float32)
        h = (g * jax.nn.sigmoid(g)) * u
        y = lax.dot_general(h, w2_ref[0], (((1,), (0,)), ((), ())),
                            preferred_element_type=jnp.float32)
        y_ref[...] = y * wr_ref[...]


def _ffn(be, act, xg, w1, w3, w2, wr):
    grid_spec = pltpu.PrefetchScalarGridSpec(
        num_scalar_prefetch=2,
        grid=(G,),
        in_specs=[
            pl.BlockSpec((BM, DM), lambda j, be, act: (j, 0)),
            pl.BlockSpec((1, DM, DF), lambda j, be, act: (be[j], 0, 0)),
            pl.BlockSpec((1, DM, DF), lambda j, be, act: (be[j], 0, 0)),
            pl.BlockSpec((1, DF, DM), lambda j, be, act: (be[j], 0, 0)),
            pl.BlockSpec((BM, 1), lambda j, be, act: (j, 0)),
        ],
        out_specs=pl.BlockSpec((BM, DM), lambda j, be, act: (j, 0)),
    )
    return pl.pallas_call(
        _ffn_body,
        grid_spec=grid_spec,
        out_shape=jax.ShapeDtypeStruct((R, DM), jnp.float32),
        compiler_params=pltpu.CompilerParams(
            dimension_semantics=("arbitrary",),
            vmem_limit_bytes=100 * 1024 * 1024),
        interpret=False,
    )(be, act, xg, w1, w3, w2, wr)


_SC_MESH = plsc.VectorSubcoreMesh(core_axis_name="c", subcore_axis_name="s")
_RPW = R // NW
_TPW = T // NW


@functools.partial(
    pl.kernel,
    mesh=_SC_MESH,
    out_type=[
        jax.ShapeDtypeStruct((R, DM), jnp.float32),
        jax.ShapeDtypeStruct((R,), jnp.float32),
    ],
    scratch_types=[
        pltpu.VMEM((T,), jnp.int32),
        pltpu.VMEM((T,), jnp.int32),
        pltpu.VMEM((T,), jnp.float32),
        pltpu.VMEM((T,), jnp.float32),
        pltpu.VMEM((R,), jnp.int32),
        pltpu.VMEM((R,), jnp.float32),
        pltpu.VMEM((_RPW // 2,), jnp.int32),
        pltpu.VMEM((_RPW // 2, DM), jnp.float32),
        pltpu.SemaphoreType.DMA,
    ],
    compiler_params=pltpu.CompilerParams(needs_layout_passes=False),
)
def _sc_dispatch(r0_h, r1_h, w0_h, w1_h, x_h, xg_h, wrow_h,
                 r0v, r1v, w0v, w1v, tokarr, warr, idxb, gbuf, sem):
    wid = lax.axis_index("s") * NC + lax.axis_index("c")
    pltpu.sync_copy(r0_h, r0v)
    pltpu.sync_copy(r1_h, r1v)
    pltpu.sync_copy(w0_h, w0v)
    pltpu.sync_copy(w1_h, w1v)
    zi = jnp.zeros((16,), jnp.int32)
    zf = jnp.zeros((16,), jnp.float32)

    def initb(i, _):
        tokarr[pl.ds(i * 16, 16)] = zi
        warr[pl.ds(i * 16, 16)] = zf
        return ()
    lax.fori_loop(0, R // 16, initb, ())

    lane = jnp.arange(16, dtype=jnp.int32)

    def scat(i, _):
        t16 = i * 16 + lane
        idx0 = r0v[pl.ds(i * 16, 16)]
        plsc.store_scatter(tokarr, [idx0], t16)
        plsc.store_scatter(warr, [idx0], w0v[pl.ds(i * 16, 16)])
        idx1 = r1v[pl.ds(i * 16, 16)]
        plsc.store_scatter(tokarr, [idx1], t16)
        plsc.store_scatter(warr, [idx1], w1v[pl.ds(i * 16, 16)])
        return ()
    lax.fori_loop(0, T // 16, scat, ())

    @pl.when(wid == 0)
    def _():
        pltpu.sync_copy(warr, wrow_h)

    base = wid * _RPW
    half = _RPW // 2
    for cchunk in range(2):
        start = base + cchunk * half
        for i in range(half // 16):
            idxb[pl.ds(i * 16, 16)] = tokarr[pl.ds(start + i * 16, 16)]
        pltpu.async_copy(x_h.at[idxb], gbuf, sem).wait()
        pltpu.sync_copy(gbuf, xg_h.at[pl.ds(start, half)])


@functools.partial(
    pl.kernel,
    mesh=_SC_MESH,
    out_type=jax.ShapeDtypeStruct((T, DM), jnp.float32),
    scratch_types=[
        pltpu.VMEM((_TPW,), jnp.int32),
        pltpu.VMEM((_TPW,), jnp.int32),
        pltpu.VMEM((_TPW, DM), jnp.float32),
        pltpu.VMEM((_TPW, DM), jnp.float32),
        pltpu.SemaphoreType.DMA,
    ],
    compiler_params=pltpu.CompilerParams(needs_layout_passes=False),
)
def _sc_combine(y_h, r0_h, r1_h, out_h, i0, i1, A, B, sem):
    wid = lax.axis_index("s") * NC + lax.axis_index("c")
    base = wid * _TPW
    pltpu.sync_copy(r0_h.at[pl.ds(base, _TPW)], i0)
    pltpu.sync_copy(r1_h.at[pl.ds(base, _TPW)], i1)
    cpa = pltpu.async_copy(y_h.at[i0], A, sem)
    cpb = pltpu.async_copy(y_h.at[i1], B, sem)
    cpa.wait()
    cpb.wait()

    def addcol(jv, _):
        col = pl.ds(jv * 16, 16)
        for irow in range(_TPW):
            A[irow, col] = A[irow, col] + B[irow, col]
        return ()
    lax.fori_loop(0, DM // 16, addcol, ())
    pltpu.sync_copy(A, out_h.at[pl.ds(base, _TPW)])


def kernel(x, router_w, w1, w2, w3):
    b, s, d = x.shape
    x2 = x.reshape(s * b, d)
    rows, wp, counts, aux, z = _router(x2, router_w)
    r0 = rows[:, 0]
    r1 = rows[:, 1]
    c = counts[0]
    nb = jnp.ceil(c * (1.0 / BM)).astype(jnp.int32)
    nbc = jnp.cumsum(nb)
    total = nbc[-1]
    jj = jnp.arange(G, dtype=jnp.int32)
    act = (jj < total).astype(jnp.int32)
    jcl = jnp.minimum(jj, total - 1)
    be = jnp.sum((nbc[None, :] <= jcl[:, None]).astype(jnp.int32), axis=1)
    be = jnp.minimum(be, NE - 1)
    xg, w_row = _sc_dispatch(r0, r1, wp[:, 0], wp[:, 1], x2)
    y = _ffn(be, act, xg, w1, w3, w2, w_row[:, None])
    out = _sc_combine(y, r0, r1)
    return out.reshape(b, s, d), aux.reshape(()), z.reshape(())


# speedup vs baseline: 1.5386x; 1.5386x over previous
import functools

import jax
import jax.numpy as jnp
from jax import lax
from jax.experimental import pallas as pl
from jax.experimental.pallas import tpu as pltpu
from jax.experimental.pallas import tpu_sc as plsc

NE = 8
DM = 768
DF = 3072
T = 2048
BM = 256
G = 23
R = 6144
NC = 2
NS = 16
NW = NC * NS
AUX_COEF = 0.01
Z_COEF = 0.001


def _router_body(x_ref, rw_ref, r0_ref, r1_ref, wp0_ref, wp1_ref,
                 counts_ref, aux_ref, z_ref):
    x = x_ref[...]
    rw = rw_ref[...]
    l = lax.dot_general(x, rw, (((1,), (1,)), ((), ())),
                        preferred_element_type=jnp.float32)
    iota_e = lax.broadcasted_iota(jnp.int32, (T, NE), 1)
    m1 = jnp.max(l, axis=1, keepdims=True)
    i1 = jnp.min(jnp.where(l == m1, iota_e, NE), axis=1, keepdims=True)
    one1 = (iota_e == i1)
    lm = jnp.where(one1, -jnp.inf, l)
    m2 = jnp.max(lm, axis=1, keepdims=True)
    i2 = jnp.min(jnp.where(lm == m2, iota_e, NE), axis=1, keepdims=True)
    one2 = (iota_e == i2)
    w0 = 1.0 / (1.0 + jnp.exp(m2 - m1))
    w1v = 1.0 - w0
    A = one1.astype(jnp.float32) + one2.astype(jnp.float32)
    r_i = lax.broadcasted_iota(jnp.int32, (T, T), 0)
    c_i = lax.broadcasted_iota(jnp.int32, (T, T), 1)
    tri = (c_i < r_i).astype(jnp.float32)
    S = lax.dot_general(tri, A, (((1,), (0,)), ((), ())),
                        preferred_element_type=jnp.float32)
    counts = jnp.sum(A, axis=0, keepdims=True)
    pc = jnp.ceil(counts * (1.0 / BM)) * BM
    e_r = lax.broadcasted_iota(jnp.int32, (NE, NE), 0)
    e_c = lax.broadcasted_iota(jnp.int32, (NE, NE), 1)
    tri8 = (e_r < e_c).astype(jnp.float32)
    off = lax.dot_general(pc, tri8, (((1,), (0,)), ((), ())),
                          preferred_element_type=jnp.float32)
    off0 = jnp.sum(off * one1, axis=1, keepdims=True)
    off1 = jnp.sum(off * one2, axis=1, keepdims=True)
    pos0 = jnp.sum(S * one1, axis=1, keepdims=True)
    pos1 = jnp.sum(S * one2, axis=1, keepdims=True)
    row0 = (off0 + pos0).astype(jnp.int32)
    row1 = (off1 + pos1).astype(jnp.int32)
    r0_ref[...] = row0
    r1_ref[...] = row1
    wp0_ref[...] = w0
    wp1_ref[...] = w1v
    counts_ref[...] = counts
    p = jnp.exp(l - m1)
    p = p / jnp.sum(p, axis=1, keepdims=True)
    imp = jnp.mean(p, axis=0, keepdims=True)
    load = jnp.mean(one1.astype(jnp.float32), axis=0, keepdims=True)
    aux_ref[...] = (NE * AUX_COEF) * jnp.sum(imp * load, keepdims=True).reshape(1, 1)
    z_ref[...] = Z_COEF * jnp.mean(l * l, keepdims=True).reshape(1, 1)


def _router(x2, rw):
    return pl.pallas_call(
        _router_body,
        out_shape=[
            jax.ShapeDtypeStruct((T, 1), jnp.int32),
            jax.ShapeDtypeStruct((T, 1), jnp.int32),
            jax.ShapeDtypeStruct((T, 1), jnp.float32),
            jax.ShapeDtypeStruct((T, 1), jnp.float32),
            jax.ShapeDtypeStruct((1, NE), jnp.float32),
            jax.ShapeDtypeStruct((1, 1), jnp.float32),
            jax.ShapeDtypeStruct((1, 1), jnp.float32),
        ],
        interpret=False,
    )(x2, rw)


NF = 1
DFC = DF // NF


def _ffn_body(be_ref, act_ref, xg_ref, w1_ref, w3_ref, w2_ref, wr_ref, y_ref):
    j = pl.program_id(0)
    f = pl.program_id(1)

    @pl.when(act_ref[j] > 0)
    def _():
        xb = xg_ref[...]
        g = lax.dot_general(xb, w1_ref[0], (((1,), (0,)), ((), ())),
                            preferred_element_type=jnp.float32)
        u = lax.dot_general(xb, w3_ref[0], (((1,), (0,)), ((), ())),
                            preferred_element_type=jnp.float32)
        h = (g * jax.nn.sigmoid(g)) * u
        y = lax.dot_general(h, w2_ref[0], (((1,), (0,)), ((), ())),
                            preferred_element_type=jnp.float32)
        y = y * wr_ref[...]

        @pl.when(f == 0)
        def _():
            y_ref[...] = y

        @pl.when(f > 0)
        def _():
            y_ref[...] += y


def _ffn(be, act, xg, w1, w3, w2, wr):
    grid_spec = pltpu.PrefetchScalarGridSpec(
        num_scalar_prefetch=2,
        grid=(G, NF),
        in_specs=[
            pl.BlockSpec((BM, DM), lambda j, f, be, act: (j, 0)),
            pl.BlockSpec((1, DM, DFC), lambda j, f, be, act: (be[j], 0, f)),
            pl.BlockSpec((1, DM, DFC), lambda j, f, be, act: (be[j], 0, f)),
            pl.BlockSpec((1, DFC, DM), lambda j, f, be, act: (be[j], f, 0)),
            pl.BlockSpec((BM, 1), lambda j, f, be, act: (j, 0)),
        ],
        out_specs=pl.BlockSpec((BM, DM), lambda j, f, be, act: (j, 0)),
    )
    return pl.pallas_call(
        _ffn_body,
        grid_spec=grid_spec,
        out_shape=jax.ShapeDtypeStruct((R, DM), jnp.float32),
        compiler_params=pltpu.CompilerParams(
            dimension_semantics=("arbitrary", "arbitrary"),
            vmem_limit_bytes=100 * 1024 * 1024),
        interpret=False,
    )(be, act, xg, w1, w3, w2, wr)


_SC_MESH = plsc.VectorSubcoreMesh(core_axis_name="c", subcore_axis_name="s")
_RPW = R // NW
_TPW = T // NW


@functools.partial(
    pl.kernel,
    mesh=_SC_MESH,
    out_type=[
        jax.ShapeDtypeStruct((R, DM), jnp.float32),
        jax.ShapeDtypeStruct((R,), jnp.float32),
    ],
    scratch_types=[
        pltpu.VMEM((T,), jnp.int32),
        pltpu.VMEM((T,), jnp.int32),
        pltpu.VMEM((T,), jnp.float32),
        pltpu.VMEM((T,), jnp.float32),
        pltpu.VMEM((R,), jnp.float32),
        pltpu.VMEM((_TPW,), jnp.int32),
        pltpu.VMEM((_TPW,), jnp.int32),
        pltpu.VMEM((_TPW, DM), jnp.float32),
        pltpu.SemaphoreType.DMA,
        pltpu.SemaphoreType.DMA,
        pltpu.SemaphoreType.DMA,
        pltpu.SemaphoreType.DMA,
        pltpu.SemaphoreType.DMA,
        pltpu.SemaphoreType.DMA,
        pltpu.SemaphoreType.DMA,
    ],
    compiler_params=pltpu.CompilerParams(needs_layout_passes=False),
)
def _sc_dispatch(r0_h, r1_h, w0_h, w1_h, x_h, xg_h, wrow_h,
                 r0full, r1full, w0full, w1full, warr,
                 ir0, ir1, xbuf, sl, s0, s1, sa, sb, sc2, sd):
    wid = lax.axis_index("s") * NC + lax.axis_index("c")
    tb = wid * _TPW
    with jax.named_scope("disp_scatter"):
        cx = pltpu.async_copy(x_h.at[pl.ds(tb, _TPW)], xbuf, sl)
        pltpu.sync_copy(r0_h.at[pl.ds(tb, _TPW)], ir0)
        pltpu.sync_copy(r1_h.at[pl.ds(tb, _TPW)], ir1)
        cx.wait()
        c0 = pltpu.async_copy(xbuf, xg_h.at[ir0], s0)
        c1 = pltpu.async_copy(xbuf, xg_h.at[ir1], s1)

    with jax.named_scope("disp_wrow"):
        @pl.when(wid == 0)
        def _():
            la = pltpu.async_copy(r0_h, r0full, sa)
            lb = pltpu.async_copy(r1_h, r1full, sb)
            lc = pltpu.async_copy(w0_h, w0full, sc2)
            ld = pltpu.async_copy(w1_h, w1full, sd)
            la.wait()
            lb.wait()
            lc.wait()
            ld.wait()

            def scat(i, _):
                idx0 = r0full[pl.ds(i * 16, 16)]
                plsc.store_scatter(warr, [idx0], w0full[pl.ds(i * 16, 16)])
                idx1 = r1full[pl.ds(i * 16, 16)]
                plsc.store_scatter(warr, [idx1], w1full[pl.ds(i * 16, 16)])
                return ()
            lax.fori_loop(0, T // 16, scat, ())
            pltpu.sync_copy(warr, wrow_h)

    with jax.named_scope("disp_drain"):
        c0.wait()
        c1.wait()


@functools.partial(
    pl.kernel,
    mesh=_SC_MESH,
    out_type=jax.ShapeDtypeStruct((T, DM), jnp.float32),
    scratch_types=[
        pltpu.VMEM((_TPW // 2,), jnp.int32),
        pltpu.VMEM((_TPW // 2,), jnp.int32),
        pltpu.VMEM((_TPW // 2,), jnp.int32),
        pltpu.VMEM((_TPW // 2,), jnp.int32),
        pltpu.VMEM((_TPW, DM), jnp.float32),
        pltpu.VMEM((_TPW, DM), jnp.float32),
        pltpu.SemaphoreType.DMA,
        pltpu.SemaphoreType.DMA,
        pltpu.SemaphoreType.DMA,
        pltpu.SemaphoreType.DMA,
        pltpu.SemaphoreType.DMA,
        pltpu.SemaphoreType.DMA,
    ],
    compiler_params=pltpu.CompilerParams(needs_layout_passes=False),
)
def _sc_combine(y_h, r0_h, r1_h, out_h, i0a, i1a, i0b, i1b, A, B,
                s0, s1, s2, s3, sw0, sw1):
    wid = lax.axis_index("s") * NC + lax.axis_index("c")
    base = wid * _TPW
    half = _TPW // 2
    pltpu.sync_copy(r0_h.at[pl.ds(base, half)], i0a)
    pltpu.sync_copy(r1_h.at[pl.ds(base, half)], i1a)
    pltpu.sync_copy(r0_h.at[pl.ds(base + half, half)], i0b)
    pltpu.sync_copy(r1_h.at[pl.ds(base + half, half)], i1b)
    ga0 = pltpu.async_copy(y_h.at[i0a], A.at[pl.ds(0, half)], s0)
    gb0 = pltpu.async_copy(y_h.at[i1a], B.at[pl.ds(0, half)], s1)
    ga1 = pltpu.async_copy(y_h.at[i0b], A.at[pl.ds(half, half)], s2)
    gb1 = pltpu.async_copy(y_h.at[i1b], B.at[pl.ds(half, half)], s3)

    def addcol(lo):
        def body(jv, _):
            col = pl.ds(jv * 16, 16)
            for irow in range(lo, lo + half):
                A[irow, col] = A[irow, col] + B[irow, col]
            return ()
        lax.fori_loop(0, DM // 16, body, ())

    ga0.wait()
    gb0.wait()
    addcol(0)
    w0 = pltpu.async_copy(A.at[pl.ds(0, half)],
                          out_h.at[pl.ds(base, half)], sw0)
    ga1.wait()
    gb1.wait()
    addcol(half)
    w1 = pltpu.async_copy(A.at[pl.ds(half, half)],
                          out_h.at[pl.ds(base + half, half)], sw1)
    w0.wait()
    w1.wait()


def kernel(x, router_w, w1, w2, w3):
    b, s, d = x.shape
    x2 = x.reshape(s * b, d)
    r02, r12, wp02, wp12, counts, aux, z = _router(x2, router_w)
    r0 = r02.reshape(T)
    r1 = r12.reshape(T)
    c = counts[0]
    nb = jnp.ceil(c * (1.0 / BM)).astype(jnp.int32)
    nbc = jnp.cumsum(nb)
    total = nbc[-1]
    jj = jnp.arange(G, dtype=jnp.int32)
    act = (jj < total).astype(jnp.int32)
    jcl = jnp.minimum(jj, total - 1)
    be = jnp.sum((nbc[None, :] <= jcl[:, None]).astype(jnp.int32), axis=1)
    be = jnp.minimum(be, NE - 1)
    xg, w_row = _sc_dispatch(r0, r1, wp02.reshape(T), wp12.reshape(T), x2)
    y = _ffn(be, act, xg, w1, w3, w2, w_row[:, None])
    out = _sc_combine(y, r0, r1)
    return out.reshape(b, s, d), aux.reshape(()), z.reshape(())
